# baseline (speedup 1.0000x reference)
import functools
import math

import jax
import jax.numpy as jnp
import numpy as np
from jax.experimental import pallas as pl
from jax.experimental.pallas import tpu as pltpu

_N = 8192
_D = 64
_A = 32
_RB = 256
_PB = 1024

_INV_SQRT_A = np.float32(math.log2(math.e) / math.sqrt(_A))
_NEG = np.float32(-1e9)
_SLOPE = np.float32(0.2)


def _proj_kernel(x_ref, W_ref, Wq_ref, Wk_ref, b_ref, h_ref, q_ref, kT_ref):
    x = x_ref[...]
    h_ref[...] = (
        jnp.dot(x, W_ref[...], preferred_element_type=jnp.float32) + b_ref[...]
    )
    q_ref[...] = (
        jnp.dot(x, Wq_ref[...], preferred_element_type=jnp.float32) * _INV_SQRT_A
    )
    kT_ref[...] = jax.lax.dot_general(
        Wk_ref[...], x, (((0,), (1,)), ((), ())),
        preferred_element_type=jnp.float32,
    )


def _project(x, W, Wq, Wk, b):
    grid = (_N // _PB,)
    return pl.pallas_call(
        _proj_kernel,
        grid=grid,
        in_specs=[
            pl.BlockSpec((_PB, _D), lambda i: (i, 0)),
            pl.BlockSpec((_D, _D), lambda i: (0, 0)),
            pl.BlockSpec((_D, _A), lambda i: (0, 0)),
            pl.BlockSpec((_D, _A), lambda i: (0, 0)),
            pl.BlockSpec((1, _D), lambda i: (0, 0)),
        ],
        out_specs=[
            pl.BlockSpec((_PB, _D), lambda i: (i, 0)),
            pl.BlockSpec((_PB, _A), lambda i: (i, 0)),
            pl.BlockSpec((_A, _PB), lambda i: (0, i)),
        ],
        out_shape=[
            jax.ShapeDtypeStruct((_N, _D), jnp.float32),
            jax.ShapeDtypeStruct((_N, _A), jnp.float32),
            jax.ShapeDtypeStruct((_A, _N), jnp.float32),
        ],
    )(x, W, Wq, Wk, b.reshape(1, _D))


def _softmax_tail(e, mask, x, h, C_ref, o_ref, relu):
    return _softmax_tail_pre(jnp.where(mask, e, _NEG), x, h, C_ref, o_ref, relu)


def _softmax_tail_pre(e, x, h, C_ref, o_ref, relu):
    m = jnp.max(e, axis=1, keepdims=True)
    p = jnp.exp2(e - m)
    s = jnp.sum(p, axis=1, keepdims=True)
    C = p * (jnp.float32(1.0) / s)
    C_ref[...] = C
    o = x + jnp.dot(C, h, preferred_element_type=jnp.float32)
    if relu:
        o = jnp.maximum(o, jnp.float32(0.0))
    o_ref[...] = o


def _load_consts(i, kT_hbm, h_hbm, kT_v, h_v, sem1, sem2):
    @pl.when(i == 0)
    def _():
        c1 = pltpu.make_async_copy(kT_hbm, kT_v, sem1)
        c2 = pltpu.make_async_copy(h_hbm, h_v, sem2)
        c1.start()
        c2.start()
        c1.wait()
        c2.wait()


def _att1_kernel(x_ref, g_ref, q_ref, kT_hbm, h_hbm, P_ref, C_ref, o_ref, w_ref,
                 kT_v, h_v, sem1, sem2):
    i = pl.program_id(0)
    _load_consts(i, kT_hbm, h_hbm, kT_v, h_v, sem1, sem2)
    e = jnp.dot(q_ref[...], kT_v[...], preferred_element_type=jnp.float32)
    e = jnp.maximum(e, e * _SLOPE)
    rows = jax.lax.broadcasted_iota(jnp.int32, (_RB, _N), 0)
    cols = jax.lax.broadcasted_iota(jnp.int32, (_RB, _N), 1)
    mask = (g_ref[...] > 0.99) | ((cols - rows) == i * _RB)
    w_ref[...] = jnp.dot(P_ref[...], mask.astype(jnp.float32),
                         preferred_element_type=jnp.float32)
    _softmax_tail(e, mask, x_ref[...], h_v[...], C_ref, o_ref, relu=True)


def _att2_kernel(x_ref, w_ref, q_ref, kT_hbm, h_hbm, C_ref, o_ref,
                 kT_v, h_v, sem1, sem2):
    i = pl.program_id(0)
    _load_consts(i, kT_hbm, h_hbm, kT_v, h_v, sem1, sem2)
    e = jnp.dot(q_ref[...], kT_v[...], preferred_element_type=jnp.float32)
    e = jnp.maximum(e, e * _SLOPE)
    wi = w_ref[...].astype(jnp.int32)
    wb = jnp.broadcast_to(wi[:, None, :], (_RB // 16, 16, _N))
    bitsel = jnp.int32(1) << jax.lax.broadcasted_iota(jnp.int32, (1, 16, 1), 1)
    mask = (wb & bitsel).reshape(_RB, _N) != 0
    _softmax_tail(e, mask, x_ref[...], h_v[...], C_ref, o_ref, relu=False)


def _att_layer1(x, graph, h, q, kT):
    grid = (_N // _RB,)
    P = np.zeros((_RB // 16, _RB), np.float32)
    for g in range(_RB // 16):
        for s in range(16):
            P[g, 16 * g + s] = float(1 << s)
    C, o, w = pl.pallas_call(
        _att1_kernel,
        grid=grid,
        in_specs=[
            pl.BlockSpec((_RB, _D), lambda i: (i, 0)),
            pl.BlockSpec((_RB, _N), lambda i: (i, 0)),
            pl.BlockSpec((_RB, _A), lambda i: (i, 0)),
            pl.BlockSpec(memory_space=pl.ANY),
            pl.BlockSpec(memory_space=pl.ANY),
            pl.BlockSpec((_RB // 16, _RB), lambda i: (0, 0)),
        ],
        out_specs=[
            pl.BlockSpec((_RB, _N), lambda i: (i, 0)),
            pl.BlockSpec((_RB, _D), lambda i: (i, 0)),
            pl.BlockSpec((_RB // 16, _N), lambda i: (i, 0)),
        ],
        out_shape=[
            jax.ShapeDtypeStruct((_N, _N), jnp.float32),
            jax.ShapeDtypeStruct((_N, _D), jnp.float32),
            jax.ShapeDtypeStruct((_N // 16, _N), jnp.float32),
        ],
        scratch_shapes=[
            pltpu.VMEM((_A, _N), jnp.float32),
            pltpu.VMEM((_N, _D), jnp.float32),
            pltpu.SemaphoreType.DMA,
            pltpu.SemaphoreType.DMA,
        ],
        compiler_params=pltpu.CompilerParams(
            vmem_limit_bytes=100 * 1024 * 1024,
        ),
    )(x, graph, q, kT, h, P)
    return C, o, w


def _att_layer2(x, w, h, q, kT):
    grid = (_N // _RB,)
    C, o = pl.pallas_call(
        _att2_kernel,
        grid=grid,
        in_specs=[
            pl.BlockSpec((_RB, _D), lambda i: (i, 0)),
            pl.BlockSpec((_RB // 16, _N), lambda i: (i, 0)),
            pl.BlockSpec((_RB, _A), lambda i: (i, 0)),
            pl.BlockSpec(memory_space=pl.ANY),
            pl.BlockSpec(memory_space=pl.ANY),
        ],
        out_specs=[
            pl.BlockSpec((_RB, _N), lambda i: (i, 0)),
            pl.BlockSpec((_RB, _D), lambda i: (i, 0)),
        ],
        out_shape=[
            jax.ShapeDtypeStruct((_N, _N), jnp.float32),
            jax.ShapeDtypeStruct((_N, _D), jnp.float32),
        ],
        scratch_shapes=[
            pltpu.VMEM((_A, _N), jnp.float32),
            pltpu.VMEM((_N, _D), jnp.float32),
            pltpu.SemaphoreType.DMA,
            pltpu.SemaphoreType.DMA,
        ],
        compiler_params=pltpu.CompilerParams(
            vmem_limit_bytes=100 * 1024 * 1024,
        ),
    )(x, w, q, kT, h)
    return C, o


def kernel(x, graph, W1, Wq1, Wk1, b1, W2, Wq2, Wk2, b2):
    h1, q1, kT1 = _project(x, W1, Wq1, Wk1, b1)
    C1, x1, w = _att_layer1(x, graph, h1, q1, kT1)
    h2, q2, kT2 = _project(x1, W2, Wq2, Wk2, b2)
    C2, x2 = _att_layer2(x1, w, h2, q2, kT2)
    return (x2, C1, C2)


# speedup vs baseline: 1.0192x; 1.0192x over previous
import math

import jax
import jax.numpy as jnp
import numpy as np
from jax.experimental import pallas as pl
from jax.experimental.pallas import tpu as pltpu

_N = 8192
_D = 64
_A = 32
_RB = 256

_INV_SQRT_A = np.float32(math.log2(math.e) / math.sqrt(_A))
_NEG = np.float32(-1e9)
_SLOPE = np.float32(0.2)


def _prologue(i, x_hbm, W_ref, Wk_ref, b_ref, x_v, kT_v, h_v, sem):
    @pl.when(i == 0)
    def _():
        cp = pltpu.make_async_copy(x_hbm, x_v, sem)
        cp.start()
        cp.wait()
        xw = x_v[...]
        h_v[...] = (
            jnp.dot(xw, W_ref[...], preferred_element_type=jnp.float32)
            + b_ref[...]
        )
        kT_v[...] = jax.lax.dot_general(
            Wk_ref[...], xw, (((0,), (1,)), ((), ())),
            preferred_element_type=jnp.float32,
        )


def _softmax_tail(e, mask, x, h, C_ref, o_ref, relu):
    e = jnp.where(mask, e, _NEG)
    m = jnp.max(e, axis=1, keepdims=True)
    p = jnp.exp2(e - m)
    s = jnp.sum(p, axis=1, keepdims=True)
    C = p * (jnp.float32(1.0) / s)
    C_ref[...] = C
    o = x + jnp.dot(C, h, preferred_element_type=jnp.float32)
    if relu:
        o = jnp.maximum(o, jnp.float32(0.0))
    o_ref[...] = o


def _att1_kernel(g_ref, x_hbm, W_ref, Wq_ref, Wk_ref, b_ref, P_ref,
                 C_ref, o_ref, w_ref, x_v, kT_v, h_v, sem):
    i = pl.program_id(0)
    _prologue(i, x_hbm, W_ref, Wk_ref, b_ref, x_v, kT_v, h_v, sem)
    xb = x_v[pl.ds(i * _RB, _RB), :]
    q = jnp.dot(xb, Wq_ref[...], preferred_element_type=jnp.float32) * _INV_SQRT_A
    e = jnp.dot(q, kT_v[...], preferred_element_type=jnp.float32)
    e = jnp.maximum(e, e * _SLOPE)
    rows = jax.lax.broadcasted_iota(jnp.int32, (_RB, _N), 0)
    cols = jax.lax.broadcasted_iota(jnp.int32, (_RB, _N), 1)
    mask = (g_ref[...] > 0.99) | ((cols - rows) == i * _RB)
    w_ref[...] = jnp.dot(P_ref[...], mask.astype(jnp.float32),
                         preferred_element_type=jnp.float32)
    _softmax_tail(e, mask, xb, h_v[...], C_ref, o_ref, relu=True)


def _att2_kernel(w_ref, x_hbm, W_ref, Wq_ref, Wk_ref, b_ref,
                 C_ref, o_ref, x_v, kT_v, h_v, sem):
    i = pl.program_id(0)
    _prologue(i, x_hbm, W_ref, Wk_ref, b_ref, x_v, kT_v, h_v, sem)
    xb = x_v[pl.ds(i * _RB, _RB), :]
    q = jnp.dot(xb, Wq_ref[...], preferred_element_type=jnp.float32) * _INV_SQRT_A
    e = jnp.dot(q, kT_v[...], preferred_element_type=jnp.float32)
    e = jnp.maximum(e, e * _SLOPE)
    wi = w_ref[...].astype(jnp.int32)
    wb = jnp.broadcast_to(wi[:, None, :], (_RB // 16, 16, _N))
    bitsel = jnp.int32(1) << jax.lax.broadcasted_iota(jnp.int32, (1, 16, 1), 1)
    mask = (wb & bitsel).reshape(_RB, _N) != 0
    _softmax_tail(e, mask, xb, h_v[...], C_ref, o_ref, relu=False)


def _scratches():
    return [
        pltpu.VMEM((_N, _D), jnp.float32),
        pltpu.VMEM((_A, _N), jnp.float32),
        pltpu.VMEM((_N, _D), jnp.float32),
        pltpu.SemaphoreType.DMA,
    ]


def _att_layer1(x, graph, W, Wq, Wk, b):
    grid = (_N // _RB,)
    P = np.zeros((_RB // 16, _RB), np.float32)
    for g in range(_RB // 16):
        for s in range(16):
            P[g, 16 * g + s] = float(1 << s)
    C, o, w = pl.pallas_call(
        _att1_kernel,
        grid=grid,
        in_specs=[
            pl.BlockSpec((_RB, _N), lambda i: (i, 0)),
            pl.BlockSpec(memory_space=pl.ANY),
            pl.BlockSpec((_D, _D), lambda i: (0, 0)),
            pl.BlockSpec((_D, _A), lambda i: (0, 0)),
            pl.BlockSpec((_D, _A), lambda i: (0, 0)),
            pl.BlockSpec((1, _D), lambda i: (0, 0)),
            pl.BlockSpec((_RB // 16, _RB), lambda i: (0, 0)),
        ],
        out_specs=[
            pl.BlockSpec((_RB, _N), lambda i: (i, 0)),
            pl.BlockSpec((_RB, _D), lambda i: (i, 0)),
            pl.BlockSpec((_RB // 16, _N), lambda i: (i, 0)),
        ],
        out_shape=[
            jax.ShapeDtypeStruct((_N, _N), jnp.float32),
            jax.ShapeDtypeStruct((_N, _D), jnp.float32),
            jax.ShapeDtypeStruct((_N // 16, _N), jnp.float32),
        ],
        scratch_shapes=_scratches(),
        compiler_params=pltpu.CompilerParams(
            vmem_limit_bytes=100 * 1024 * 1024,
        ),
    )(graph, x, W, Wq, Wk, b.reshape(1, _D), P)
    return C, o, w


def _att_layer2(x, w, W, Wq, Wk, b):
    grid = (_N // _RB,)
    C, o = pl.pallas_call(
        _att2_kernel,
        grid=grid,
        in_specs=[
            pl.BlockSpec((_RB // 16, _N), lambda i: (i, 0)),
            pl.BlockSpec(memory_space=pl.ANY),
            pl.BlockSpec((_D, _D), lambda i: (0, 0)),
            pl.BlockSpec((_D, _A), lambda i: (0, 0)),
            pl.BlockSpec((_D, _A), lambda i: (0, 0)),
            pl.BlockSpec((1, _D), lambda i: (0, 0)),
        ],
        out_specs=[
            pl.BlockSpec((_RB, _N), lambda i: (i, 0)),
            pl.BlockSpec((_RB, _D), lambda i: (i, 0)),
        ],
        out_shape=[
            jax.ShapeDtypeStruct((_N, _N), jnp.float32),
            jax.ShapeDtypeStruct((_N, _D), jnp.float32),
        ],
        scratch_shapes=_scratches(),
        compiler_params=pltpu.CompilerParams(
            vmem_limit_bytes=100 * 1024 * 1024,
        ),
    )(w, x, W, Wq, Wk, b.reshape(1, _D))
    return C, o


def kernel(x, graph, W1, Wq1, Wk1, b1, W2, Wq2, Wk2, b2):
    C1, x1, w = _att_layer1(x, graph, W1, Wq1, Wk1, b1)
    C2, x2 = _att_layer2(x1, w, W2, Wq2, Wk2, b2)
    return (x2, C1, C2)


# speedup vs baseline: 1.0337x; 1.0142x over previous
import math

import jax
import jax.numpy as jnp
import numpy as np
from jax.experimental import pallas as pl
from jax.experimental.pallas import tpu as pltpu

_N = 8192
_D = 64
_A = 32
_RB = 256
_RB2 = 512

_INV_SQRT_A = np.float32(math.log2(math.e) / math.sqrt(_A))
_NEG = np.float32(-1e9)
_SLOPE = np.float32(0.2)


def _prologue(i, x_hbm, W_ref, Wk_ref, b_ref, x_v, kT_v, h_v, sem):
    @pl.when(i == 0)
    def _():
        cp = pltpu.make_async_copy(x_hbm, x_v, sem)
        cp.start()
        cp.wait()
        xw = x_v[...]
        h_v[...] = (
            jnp.dot(xw, W_ref[...], preferred_element_type=jnp.float32)
            + b_ref[...]
        )
        kT_v[...] = jax.lax.dot_general(
            Wk_ref[...], xw, (((0,), (1,)), ((), ())),
            preferred_element_type=jnp.float32,
        )


def _softmax_tail(e, mask, x, h, C_ref, o_ref, relu):
    e = jnp.where(mask, e, _NEG)
    m = jnp.max(e, axis=1, keepdims=True)
    p = jnp.exp2(e - m)
    s = jnp.sum(p, axis=1, keepdims=True)
    C = p * (jnp.float32(1.0) / s)
    C_ref[...] = C
    o = x + jnp.dot(C, h, preferred_element_type=jnp.float32)
    if relu:
        o = jnp.maximum(o, jnp.float32(0.0))
    o_ref[...] = o


def _att1_kernel(g_ref, x_hbm, W_ref, Wq_ref, Wk_ref, b_ref, P_ref,
                 C_ref, o_ref, w_ref, x_v, kT_v, h_v, sem):
    i = pl.program_id(0)
    _prologue(i, x_hbm, W_ref, Wk_ref, b_ref, x_v, kT_v, h_v, sem)
    xb = x_v[pl.ds(i * _RB, _RB), :]
    q = jnp.dot(xb, Wq_ref[...], preferred_element_type=jnp.float32) * _INV_SQRT_A
    e = jnp.dot(q, kT_v[...], preferred_element_type=jnp.float32)
    e = jnp.maximum(e, e * _SLOPE)
    rows = jax.lax.broadcasted_iota(jnp.int32, (_RB, _N), 0)
    cols = jax.lax.broadcasted_iota(jnp.int32, (_RB, _N), 1)
    mask = (g_ref[...] > 0.99) | ((cols - rows) == i * _RB)
    w_ref[...] = jnp.dot(P_ref[...], mask.astype(jnp.float32),
                         preferred_element_type=jnp.float32)
    _softmax_tail(e, mask, xb, h_v[...], C_ref, o_ref, relu=True)


def _att2_kernel(w_ref, x_hbm, W_ref, Wq_ref, Wk_ref, b_ref,
                 C_ref, o_ref, x_v, kT_v, h_v, sem):
    i = pl.program_id(0)
    _prologue(i, x_hbm, W_ref, Wk_ref, b_ref, x_v, kT_v, h_v, sem)
    xb = x_v[pl.ds(i * _RB2, _RB2), :]
    q = jnp.dot(xb, Wq_ref[...], preferred_element_type=jnp.float32) * _INV_SQRT_A
    e = jnp.dot(q, kT_v[...], preferred_element_type=jnp.float32)
    e = jnp.maximum(e, e * _SLOPE)
    wi = w_ref[...].astype(jnp.int32)
    wb = jnp.broadcast_to(wi[:, None, :], (_RB2 // 16, 16, _N))
    bitsel = jnp.int32(1) << jax.lax.broadcasted_iota(jnp.int32, (1, 16, 1), 1)
    mask = (wb & bitsel).reshape(_RB2, _N) != 0
    _softmax_tail(e, mask, xb, h_v[...], C_ref, o_ref, relu=False)


def _scratches():
    return [
        pltpu.VMEM((_N, _D), jnp.float32),
        pltpu.VMEM((_A, _N), jnp.float32),
        pltpu.VMEM((_N, _D), jnp.float32),
        pltpu.SemaphoreType.DMA,
    ]


def _att_layer1(x, graph, W, Wq, Wk, b):
    grid = (_N // _RB,)
    P = np.zeros((_RB // 16, _RB), np.float32)
    for g in range(_RB // 16):
        for s in range(16):
            P[g, 16 * g + s] = float(1 << s)
    C, o, w = pl.pallas_call(
        _att1_kernel,
        grid=grid,
        in_specs=[
            pl.BlockSpec((_RB, _N), lambda i: (i, 0)),
            pl.BlockSpec(memory_space=pl.ANY),
            pl.BlockSpec((_D, _D), lambda i: (0, 0)),
            pl.BlockSpec((_D, _A), lambda i: (0, 0)),
            pl.BlockSpec((_D, _A), lambda i: (0, 0)),
            pl.BlockSpec((1, _D), lambda i: (0, 0)),
            pl.BlockSpec((_RB // 16, _RB), lambda i: (0, 0)),
        ],
        out_specs=[
            pl.BlockSpec((_RB, _N), lambda i: (i, 0)),
            pl.BlockSpec((_RB, _D), lambda i: (i, 0)),
            pl.BlockSpec((_RB // 16, _N), lambda i: (i, 0)),
        ],
        out_shape=[
            jax.ShapeDtypeStruct((_N, _N), jnp.float32),
            jax.ShapeDtypeStruct((_N, _D), jnp.float32),
            jax.ShapeDtypeStruct((_N // 16, _N), jnp.float32),
        ],
        scratch_shapes=_scratches(),
        compiler_params=pltpu.CompilerParams(
            vmem_limit_bytes=100 * 1024 * 1024,
        ),
    )(graph, x, W, Wq, Wk, b.reshape(1, _D), P)
    return C, o, w


def _att_layer2(x, w, W, Wq, Wk, b):
    grid = (_N // _RB2,)
    C, o = pl.pallas_call(
        _att2_kernel,
        grid=grid,
        in_specs=[
            pl.BlockSpec((_RB2 // 16, _N), lambda i: (i, 0)),
            pl.BlockSpec(memory_space=pl.ANY),
            pl.BlockSpec((_D, _D), lambda i: (0, 0)),
            pl.BlockSpec((_D, _A), lambda i: (0, 0)),
            pl.BlockSpec((_D, _A), lambda i: (0, 0)),
            pl.BlockSpec((1, _D), lambda i: (0, 0)),
        ],
        out_specs=[
            pl.BlockSpec((_RB2, _N), lambda i: (i, 0)),
            pl.BlockSpec((_RB2, _D), lambda i: (i, 0)),
        ],
        out_shape=[
            jax.ShapeDtypeStruct((_N, _N), jnp.float32),
            jax.ShapeDtypeStruct((_N, _D), jnp.float32),
        ],
        scratch_shapes=_scratches(),
        compiler_params=pltpu.CompilerParams(
            vmem_limit_bytes=100 * 1024 * 1024,
        ),
    )(w, x, W, Wq, Wk, b.reshape(1, _D))
    return C, o


def kernel(x, graph, W1, Wq1, Wk1, b1, W2, Wq2, Wk2, b2):
    C1, x1, w = _att_layer1(x, graph, W1, Wq1, Wk1, b1)
    C2, x2 = _att_layer2(x1, w, W2, Wq2, Wk2, b2)
    return (x2, C1, C2)
